# baseline (speedup 1.0000x reference)
import jax
import jax.numpy as jnp
from jax.experimental import pallas as pl
from jax.experimental.pallas import tpu as pltpu

N = 131072
D = 32
K = 512
G = 8192
R = 1024
H = 256
NQ = R // H
NB = N // R
W = 96
NEG_INF = float("-inf")


def _scan_half(s, b):
    d = 1
    while d < H:
        sb = jnp.concatenate([b[H - d:], b[:H - d]], axis=0)
        okf = jnp.where(b == sb, 0.0, NEG_INF)
        ss = jnp.concatenate([s[H - d:], s[:H - d]], axis=0)
        s = jnp.maximum(s, ss + okf)
        d *= 2
    return s


def _body(bcol_ref, brow_ref, le_ref, pvt2_ref, p2_ref, out_ref, s_ref):
    i = pl.program_id(0)

    @pl.when(i == 0)
    def _init():
        out_ref[...] = jnp.full((G, K), NEG_INF, dtype=jnp.float32)

    e = le_ref[...]
    t = (jnp.dot(e, pvt2_ref[...], preferred_element_type=jnp.float32)
         - p2_ref[...])
    m = jnp.max(t, axis=1, keepdims=True)
    ssum = jnp.sum(jnp.exp(t - m), axis=1, keepdims=True)
    s = t - (m + jnp.log(ssum))

    b = bcol_ref[0]
    parts = []
    for q in range(NQ):
        bq = b[q * H:(q + 1) * H]
        sq = _scan_half(s[q * H:(q + 1) * H], bq)
        if q:
            contf = jnp.where(bq == b[q * H - 1:q * H], 0.0, NEG_INF)
            prev = parts[q - 1]
            sq = jnp.maximum(
                sq, jnp.broadcast_to(prev[H - 1:H], (H, K)) + contf)
        parts.append(sq)

    brow = brow_ref[0]
    g_first = jnp.min(brow)
    g_last = jnp.max(brow)

    wbase = jnp.minimum((g_first // 8) * 8, G - W)
    wbase = pl.multiple_of(wbase, 8)
    gv = wbase + jax.lax.broadcasted_iota(jnp.int32, (W, 1), 0)
    cnt = jnp.sum((brow <= gv).astype(jnp.int32), axis=1, keepdims=True)
    cnt_prev = jnp.concatenate([jnp.zeros((1, 1), jnp.int32), cnt[:W - 1]], axis=0)
    valid = cnt > cnt_prev
    ii1 = jax.lax.broadcasted_iota(jnp.int32, (1, H), 1)
    buf = jnp.zeros((W, K), jnp.float32)
    for q in range(NQ):
        ohq = ((ii1 == cnt - 1 - q * H) & valid).astype(jnp.float32)
        buf = buf + jnp.dot(ohq, parts[q], preferred_element_type=jnp.float32)
    buf = jnp.where(valid, buf, NEG_INF)

    in_window = g_last < wbase + W

    @pl.when(in_window)
    def _vec():
        cur = out_ref[pl.ds(wbase, W), :]
        out_ref[pl.ds(wbase, W), :] = jnp.maximum(cur, buf)

    @pl.when(jnp.logical_not(in_window))
    def _fallback():
        for q in range(NQ):
            s_ref[q * H:(q + 1) * H] = parts[q]

        def upd(g, cp):
            cc = jnp.sum(jnp.where(brow <= g, 1, 0))

            @pl.when(cc > cp)
            def _():
                row = s_ref[pl.ds(cc - 1, 1), :]
                out_ref[pl.ds(g, 1), :] = jnp.maximum(out_ref[pl.ds(g, 1), :], row)

            return cc

        jax.lax.fori_loop(g_first, g_last + 1, upd, jnp.int32(0))

    @pl.when(i == NB - 1)
    def _fin():
        v = out_ref[...]
        out_ref[...] = jnp.where(v == NEG_INF, v, jnp.exp(v))


def kernel(le_embeddings, belonging, prototype_vectors):
    pvt2 = 2.0 * prototype_vectors.T
    p2 = jnp.sum(prototype_vectors * prototype_vectors, axis=1)[None, :]
    bcol = belonging.reshape(NB, R, 1)
    brow = belonging.reshape(NB, 1, R)
    return pl.pallas_call(
        _body,
        grid=(NB,),
        in_specs=[
            pl.BlockSpec((1, R, 1), lambda i: (i, 0, 0)),
            pl.BlockSpec((1, 1, R), lambda i: (i, 0, 0)),
            pl.BlockSpec((R, D), lambda i: (i, 0)),
            pl.BlockSpec((D, K), lambda i: (0, 0)),
            pl.BlockSpec((1, K), lambda i: (0, 0)),
        ],
        out_specs=pl.BlockSpec((G, K), lambda i: (0, 0)),
        out_shape=jax.ShapeDtypeStruct((G, K), jnp.float32),
        scratch_shapes=[pltpu.VMEM((R, K), jnp.float32)],
    )(bcol, brow, le_embeddings, pvt2, p2)


# speedup vs baseline: 1.0319x; 1.0319x over previous
import jax
import jax.numpy as jnp
from jax.experimental import pallas as pl
from jax.experimental.pallas import tpu as pltpu

N = 131072
D = 32
K = 512
G = 8192
R = 1024
H = 256
NQ = R // H
NB = N // R
W = 128
NEG_INF = float("-inf")


def _scan_half(s, b):
    d = 1
    while d < H:
        sb = jnp.concatenate([b[H - d:], b[:H - d]], axis=0)
        okf = jnp.where(b == sb, 0.0, NEG_INF)
        ss = jnp.concatenate([s[H - d:], s[:H - d]], axis=0)
        s = jnp.maximum(s, ss + okf)
        d *= 2
    return s


def _body(bcol_ref, brow_ref, le_ref, pvt2_ref, p2_ref, out_ref, s_ref):
    i = pl.program_id(0)

    @pl.when(i == 0)
    def _init():
        out_ref[...] = jnp.full((G, K), NEG_INF, dtype=jnp.float32)

    e = le_ref[...]
    t = (jnp.dot(e, pvt2_ref[...], preferred_element_type=jnp.float32)
         - p2_ref[...])
    m = jnp.max(t, axis=1, keepdims=True)
    ssum = jnp.sum(jnp.exp(t - m), axis=1, keepdims=True)
    s = t - (m + jnp.log(ssum))

    b = bcol_ref[0]
    parts = []
    for q in range(NQ):
        bq = b[q * H:(q + 1) * H]
        sq = _scan_half(s[q * H:(q + 1) * H], bq)
        if q:
            contf = jnp.where(bq == b[q * H - 1:q * H], 0.0, NEG_INF)
            prev = parts[q - 1]
            sq = jnp.maximum(
                sq, jnp.broadcast_to(prev[H - 1:H], (H, K)) + contf)
        parts.append(sq)

    brow = brow_ref[0]
    g_first = jnp.min(brow)
    g_last = jnp.max(brow)

    wbase = jnp.minimum((g_first // 8) * 8, G - W)
    wbase = pl.multiple_of(wbase, 8)
    gv = wbase + jax.lax.broadcasted_iota(jnp.int32, (W, 1), 0)
    cnt = jnp.sum((brow <= gv).astype(jnp.int32), axis=1, keepdims=True)
    cnt_prev = jnp.concatenate([jnp.zeros((1, 1), jnp.int32), cnt[:W - 1]], axis=0)
    valid = cnt > cnt_prev
    ii1 = jax.lax.broadcasted_iota(jnp.int32, (1, H), 1)
    buf = jnp.zeros((W, K), jnp.float32)
    for q in range(NQ):
        ohq = ((ii1 == cnt - 1 - q * H) & valid).astype(jnp.float32)
        buf = buf + jnp.dot(ohq, parts[q], preferred_element_type=jnp.float32)
    buf = jnp.where(valid, buf, NEG_INF)

    in_window = g_last < wbase + W

    @pl.when(in_window)
    def _vec():
        cur = out_ref[pl.ds(wbase, W), :]
        out_ref[pl.ds(wbase, W), :] = jnp.maximum(cur, buf)

    @pl.when(jnp.logical_not(in_window))
    def _fallback():
        for q in range(NQ):
            s_ref[q * H:(q + 1) * H] = parts[q]

        def upd(g, cp):
            cc = jnp.sum(jnp.where(brow <= g, 1, 0))

            @pl.when(cc > cp)
            def _():
                row = s_ref[pl.ds(cc - 1, 1), :]
                out_ref[pl.ds(g, 1), :] = jnp.maximum(out_ref[pl.ds(g, 1), :], row)

            return cc

        jax.lax.fori_loop(g_first, g_last + 1, upd, jnp.int32(0))

    @pl.when(i == NB - 1)
    def _fin():
        v = out_ref[...]
        out_ref[...] = jnp.where(v == NEG_INF, v, jnp.exp(v))


def kernel(le_embeddings, belonging, prototype_vectors):
    pvt2 = 2.0 * prototype_vectors.T
    p2 = jnp.sum(prototype_vectors * prototype_vectors, axis=1)[None, :]
    bcol = belonging.reshape(NB, R, 1)
    brow = belonging.reshape(NB, 1, R)
    return pl.pallas_call(
        _body,
        grid=(NB,),
        in_specs=[
            pl.BlockSpec((1, R, 1), lambda i: (i, 0, 0)),
            pl.BlockSpec((1, 1, R), lambda i: (i, 0, 0)),
            pl.BlockSpec((R, D), lambda i: (i, 0)),
            pl.BlockSpec((D, K), lambda i: (0, 0)),
            pl.BlockSpec((1, K), lambda i: (0, 0)),
        ],
        out_specs=pl.BlockSpec((G, K), lambda i: (0, 0)),
        out_shape=jax.ShapeDtypeStruct((G, K), jnp.float32),
        scratch_shapes=[pltpu.VMEM((R, K), jnp.float32)],
    )(bcol, brow, le_embeddings, pvt2, p2)
